# Initial kernel scaffold; baseline (speedup 1.0000x reference)
import jax
import jax.numpy as jnp
from jax.experimental import pallas as pl


def kernel(x, codebook):
    raise NotImplementedError("write your pallas kernel here")


# speedup vs baseline: 4.4068x; 4.4068x over previous
import functools

import jax
import jax.numpy as jnp
from jax.experimental import pallas as pl

M = 8
K = 1024
D = 256
SUB = D // M

TN = 512


def _pq_body(x_ref, cb_ref, codes_t_ref, side_ref):
    xs = x_ref[...]
    for m in range(M):
        cb = cb_ref[m]
        xm = xs[:, m * SUB:(m + 1) * SUB]
        cross = jax.lax.dot_general(
            xm, cb, (((1,), (1,)), ((), ())),
            preferred_element_type=jnp.float32)
        x_sq = jnp.sum(xm * xm, axis=1, keepdims=True)
        c_sq = jnp.sum(cb * cb, axis=1)[None, :]
        dist = x_sq - 2.0 * cross + c_sq
        dmin = jnp.min(dist, axis=1, keepdims=True)
        iota = jax.lax.broadcasted_iota(jnp.int32, dist.shape, 1)
        codes_m = jnp.min(jnp.where(dist == dmin, iota, K), axis=1)
        codes_t_ref[m, :] = codes_m
        onehot = (iota == codes_m[:, None]).astype(jnp.float32)
        side_ref[m] = jax.lax.dot_general(
            onehot, cb, (((1,), (0,)), ((), ())),
            preferred_element_type=jnp.float32)


@jax.jit
def kernel(x, codebook):
    n = x.shape[0]
    grid = n // TN
    codes_t, side = pl.pallas_call(
        _pq_body,
        grid=(grid,),
        in_specs=[
            pl.BlockSpec((TN, D), lambda i: (i, 0)),
            pl.BlockSpec((M, K, SUB), lambda i: (0, 0, 0)),
        ],
        out_specs=[
            pl.BlockSpec((M, TN), lambda i: (0, i)),
            pl.BlockSpec((M, TN, SUB), lambda i: (0, i, 0)),
        ],
        out_shape=[
            jax.ShapeDtypeStruct((M, n), jnp.int32),
            jax.ShapeDtypeStruct((M, n, SUB), jnp.float32),
        ],
    )(x, codebook)
    codes = codes_t.T
    x_recon = jnp.transpose(side, (1, 0, 2)).reshape(n, D)
    return (x_recon, codes, side)


# speedup vs baseline: 4.9597x; 1.1255x over previous
import jax
import jax.numpy as jnp
from jax.experimental import pallas as pl
from jax.experimental.pallas import tpu as pltpu

M = 8
K = 1024
D = 256
SUB = D // M

TN = 512


def _pq_body(x_ref, cb_ref, m2cb_ref, codes_t_ref, side_ref, csq_ref):
    @pl.when(pl.program_id(0) == 0)
    def _prep():
        for m in range(M):
            cb = cb_ref[m]
            csq_ref[m, :] = jnp.sum(cb * cb, axis=1)

    xs = x_ref[...]
    iota_f = jax.lax.broadcasted_iota(jnp.int32, (TN, K), 1).astype(jnp.float32)
    for m in range(M):
        xm = xs[:, m * SUB:(m + 1) * SUB]
        cross2 = jax.lax.dot_general(
            xm, m2cb_ref[m], (((1,), (1,)), ((), ())),
            preferred_element_type=jnp.float32)
        dist = cross2 + csq_ref[m, :][None, :]
        dmin = jnp.min(dist, axis=1, keepdims=True)
        idxf = jnp.min(jnp.where(dist == dmin, iota_f, float(K)), axis=1)
        codes_t_ref[m, :] = idxf.astype(jnp.int32)
        onehot = (iota_f == idxf[:, None]).astype(jnp.float32)
        side_ref[m] = jax.lax.dot_general(
            onehot, cb_ref[m], (((1,), (0,)), ((), ())),
            preferred_element_type=jnp.float32)


@jax.jit
def kernel(x, codebook):
    n = x.shape[0]
    grid = n // TN
    m2cb = -2.0 * codebook
    codes_t, side = pl.pallas_call(
        _pq_body,
        grid=(grid,),
        in_specs=[
            pl.BlockSpec((TN, D), lambda i: (i, 0)),
            pl.BlockSpec((M, K, SUB), lambda i: (0, 0, 0)),
            pl.BlockSpec((M, K, SUB), lambda i: (0, 0, 0)),
        ],
        out_specs=[
            pl.BlockSpec((M, TN), lambda i: (0, i)),
            pl.BlockSpec((M, TN, SUB), lambda i: (0, i, 0)),
        ],
        out_shape=[
            jax.ShapeDtypeStruct((M, n), jnp.int32),
            jax.ShapeDtypeStruct((M, n, SUB), jnp.float32),
        ],
        scratch_shapes=[pltpu.VMEM((M, K), jnp.float32)],
    )(x, codebook, m2cb)
    codes = codes_t.T
    x_recon = jnp.transpose(side, (1, 0, 2)).reshape(n, D)
    return (x_recon, codes, side)
